# baseline (speedup 1.0000x reference)
import functools

import jax
import jax.numpy as jnp
from jax import lax
from jax.experimental import pallas as pl
from jax.experimental.pallas import tpu as pltpu
from jax.experimental.pallas import tpu_sc as plsc

N = 100000
E = 3200000
IN_F = 16
HID = 64
OUT_F = 8

NC = 2
NS = 16
LW = 16
SUB = 128
WIN_ROWS = 5
WIN = SUB * WIN_ROWS
N_WIN = 160
EDGES_PER_TILE = WIN * N_WIN
E_PAD = EDGES_PER_TILE * NC * NS
PAD = E_PAD - E
DUMP = 352
N_PAD = N + DUMP
ROWS_PER_TILE_IDX = EDGES_PER_TILE // SUB
SLICE = N_PAD // NS

BR = 2048
NBLK = N_PAD // BR
PKR = BR * LW // 128
NPK = N_PAD * LW // 128
DGR = BR // 128
NDG = N_PAD // 128

SEGS = [(k * WIN, WIN) for k in range(SLICE // WIN)]
if SLICE % WIN:
  SEGS.append((SLICE // WIN * WIN, SLICE % WIN))


def _scatter_sc_body(n_chunks, src_hbm, dst_hbm, u_hbm, out_hbm,
                     src_v0, src_v1, dst_v0, dst_v1, rows_v0, rows_v1, agg_sh,
                     sem_g0, sem_g1, sem_s0, sem_s1):
  c = lax.axis_index("c")
  s = lax.axis_index("s")
  row_base = (c * NS + s) * ROWS_PER_TILE_IDX
  src_v = (src_v0, src_v1)
  dst_v = (dst_v0, dst_v1)
  rows_v = (rows_v0, rows_v1)
  sem_g = (sem_g0, sem_g1)
  sem_s = (sem_s0, sem_s1)

  def _gathers(ch, b):
    return [pltpu.make_async_copy(u_hbm.at[ch].at[src_v[b].at[j]],
                                  rows_v[b].at[pl.ds(j * SUB, SUB)], sem_g[b])
            for j in range(WIN_ROWS)]

  def _scatters(b):
    return [pltpu.make_async_copy(rows_v[b].at[pl.ds(j * SUB, SUB)],
                                  agg_sh.at[dst_v[b].at[j]], sem_s[b])
            for j in range(WIN_ROWS)]

  def _load_idx(b, w):
    r0 = row_base + w * WIN_ROWS
    pltpu.sync_copy(src_hbm.at[pl.ds(r0, WIN_ROWS)], src_v[b])
    pltpu.sync_copy(dst_hbm.at[pl.ds(r0, WIN_ROWS)], dst_v[b])

  for ch in range(n_chunks):
    def _z(i, _):
      rows_v0[i, :] = jnp.zeros((LW,), jnp.float32)
      return 0
    lax.fori_loop(0, WIN, _z, 0)
    zb = s * SLICE
    for off, sz in SEGS:
      pltpu.sync_copy(rows_v0.at[pl.ds(0, sz)], agg_sh.at[pl.ds(zb + off, sz)])
    plsc.subcore_barrier()

    for b in range(2):
      _load_idx(b, b)
      for cp in _gathers(ch, b):
        cp.start()

    def _pair(k, _):
      for b in range(2):
        for cp in _gathers(ch, b):
          cp.wait()
        for cp in _scatters(b):
          cp.start(add=True)
      for b in range(2):
        w = 2 * k + 2 + b

        @pl.when(w < N_WIN)
        def _():
          for cp in _scatters(b):
            cp.wait()
          _load_idx(b, w)
          for cp in _gathers(ch, b):
            cp.start()
      return 0
    lax.fori_loop(0, N_WIN // 2, _pair, 0)
    for b in range(2):
      for cp in _scatters(b):
        cp.wait()
    plsc.subcore_barrier()

    fb = s * SLICE
    for off, sz in SEGS:
      pltpu.sync_copy(agg_sh.at[pl.ds(fb + off, sz)],
                      out_hbm.at[ch, c, pl.ds(fb + off, sz)])


def _scatter64_sc_body(src_hbm, dst_hbm, u_hbm, out_hbm,
                       src_v0, src_v1, dst_v0, dst_v1, rows_v0, rows_v1,
                       agg_sh, sem_g0, sem_g1, sem_s0, sem_s1):
  c = lax.axis_index("c")
  s = lax.axis_index("s")
  row_base = (c * NS + s) * ROWS_PER_TILE_IDX
  src_v = (src_v0, src_v1)
  dst_v = (dst_v0, dst_v1)
  rows_v = (rows_v0, rows_v1)
  sem_g = (sem_g0, sem_g1)
  sem_s = (sem_s0, sem_s1)

  def _gathers(b):
    return [pltpu.make_async_copy(u_hbm.at[src_v[b].at[j]],
                                  rows_v[b].at[pl.ds(j * SUB, SUB)], sem_g[b])
            for j in range(WIN_ROWS)]

  def _scatters(b):
    return [pltpu.make_async_copy(rows_v[b].at[pl.ds(j * SUB, SUB)],
                                  agg_sh.at[dst_v[b].at[j]], sem_s[b])
            for j in range(WIN_ROWS)]

  def _load_idx(ch, b, w):
    r0 = row_base + w * WIN_ROWS
    pltpu.sync_copy(src_hbm.at[pl.ds(r0, WIN_ROWS)], src_v[b])
    for r in range(WIN_ROWS):
      for j in range(SUB // LW):
        sl = (r, pl.ds(j * LW, LW))
        src_v[b][sl] = src_v[b][sl] * 4 + ch
    pltpu.sync_copy(dst_hbm.at[pl.ds(r0, WIN_ROWS)], dst_v[b])

  for ch in range(4):
    def _z(i, _):
      rows_v0[i, :] = jnp.zeros((LW,), jnp.float32)
      return 0
    lax.fori_loop(0, WIN, _z, 0)
    zb = s * SLICE
    for off, sz in SEGS:
      pltpu.sync_copy(rows_v0.at[pl.ds(0, sz)], agg_sh.at[pl.ds(zb + off, sz)])
    plsc.subcore_barrier()

    for b in range(2):
      _load_idx(ch, b, b)
      for cp in _gathers(b):
        cp.start()

    def _pair(k, _):
      for b in range(2):
        for cp in _gathers(b):
          cp.wait()
        for cp in _scatters(b):
          cp.start(add=True)
      for b in range(2):
        w = 2 * k + 2 + b

        @pl.when(w < N_WIN)
        def _():
          for cp in _scatters(b):
            cp.wait()
          _load_idx(ch, b, w)
          for cp in _gathers(b):
            cp.start()
      return 0
    lax.fori_loop(0, N_WIN // 2, _pair, 0)
    for b in range(2):
      for cp in _scatters(b):
        cp.wait()
    plsc.subcore_barrier()

    fb = s * SLICE
    for off, sz in SEGS:
      pltpu.sync_copy(agg_sh.at[pl.ds(fb + off, sz)],
                      out_hbm.at[c, pl.ds(fb + off, sz), ch])


def _make_scatter_kernel(n_chunks):
  mesh = plsc.VectorSubcoreMesh(core_axis_name="c", subcore_axis_name="s")
  if n_chunks == 4:
    out_t = jax.ShapeDtypeStruct((NC, N_PAD, 4, LW), jnp.float32)
    body = _scatter64_sc_body
  else:
    out_t = jax.ShapeDtypeStruct((n_chunks, NC, N_PAD, LW), jnp.float32)
    body = functools.partial(_scatter_sc_body, n_chunks)
  return functools.partial(
      pl.kernel,
      out_type=out_t,
      mesh=mesh,
      scratch_types=[
          pltpu.VMEM((WIN_ROWS, SUB), jnp.int32),
          pltpu.VMEM((WIN_ROWS, SUB), jnp.int32),
          pltpu.VMEM((WIN_ROWS, SUB), jnp.int32),
          pltpu.VMEM((WIN_ROWS, SUB), jnp.int32),
          pltpu.VMEM((WIN, LW), jnp.float32),
          pltpu.VMEM((WIN, LW), jnp.float32),
          pltpu.VMEM_SHARED((N_PAD, LW), jnp.float32),
          pltpu.SemaphoreType.DMA,
          pltpu.SemaphoreType.DMA,
          pltpu.SemaphoreType.DMA,
          pltpu.SemaphoreType.DMA,
      ],
      compiler_params=pltpu.CompilerParams(use_tc_tiling_on_sc=False),
  )(body)


_scatter_c1 = _make_scatter_kernel(1)
_scatter_c4 = _make_scatter_kernel(4)


def _deg_sc_body(dst_hbm, out_hbm, dst_v0, dst_v1, ones_v, zeros_v, deg_sh,
                 sem_s0, sem_s1):
  c = lax.axis_index("c")
  s = lax.axis_index("s")
  row_base = (c * NS + s) * ROWS_PER_TILE_IDX
  dst_v = (dst_v0, dst_v1)
  sem_s = (sem_s0, sem_s1)

  def _o(i, _):
    ones_v[pl.ds(i * LW, LW)] = jnp.ones((LW,), jnp.float32)
    return 0
  lax.fori_loop(0, SUB // LW, _o, 0)

  def _z(i, _):
    zeros_v[pl.ds(i * LW, LW)] = jnp.zeros((LW,), jnp.float32)
    return 0
  lax.fori_loop(0, WIN // LW, _z, 0)

  zb = s * SLICE
  for off, sz in SEGS:
    pltpu.sync_copy(zeros_v.at[pl.ds(0, sz)], deg_sh.at[pl.ds(zb + off, sz)])
  plsc.subcore_barrier()

  def _scatters(b):
    return [pltpu.make_async_copy(ones_v, deg_sh.at[dst_v[b].at[j]], sem_s[b])
            for j in range(WIN_ROWS)]

  def _load_idx(b, w):
    r0 = row_base + w * WIN_ROWS
    pltpu.sync_copy(dst_hbm.at[pl.ds(r0, WIN_ROWS)], dst_v[b])

  for b in range(2):
    _load_idx(b, b)

  def _pair(k, _):
    for b in range(2):
      for cp in _scatters(b):
        cp.start(add=True)
    for b in range(2):
      w = 2 * k + 2 + b

      @pl.when(w < N_WIN)
      def _():
        for cp in _scatters(b):
          cp.wait()
        _load_idx(b, w)
    return 0
  lax.fori_loop(0, N_WIN // 2, _pair, 0)
  for b in range(2):
    for cp in _scatters(b):
      cp.wait()
  plsc.subcore_barrier()

  fb = s * SLICE
  for off, sz in SEGS:
    pltpu.sync_copy(deg_sh.at[pl.ds(fb + off, sz)],
                    out_hbm.at[c, 0, pl.ds(fb + off, sz)])


_deg_kernel = pl.kernel(
    _deg_sc_body,
    out_type=jax.ShapeDtypeStruct((NC, 1, N_PAD), jnp.float32),
    mesh=plsc.VectorSubcoreMesh(core_axis_name="c", subcore_axis_name="s"),
    scratch_types=[
        pltpu.VMEM((WIN_ROWS, SUB), jnp.int32),
        pltpu.VMEM((WIN_ROWS, SUB), jnp.int32),
        pltpu.VMEM((SUB,), jnp.float32),
        pltpu.VMEM((WIN,), jnp.float32),
        pltpu.VMEM_SHARED((N_PAD,), jnp.float32),
        pltpu.SemaphoreType.DMA,
        pltpu.SemaphoreType.DMA,
    ],
    compiler_params=pltpu.CompilerParams(use_tc_tiling_on_sc=False),
)


def _t1_body(x_ref, degp_ref, norm_ref, u_ref):
  deg = degp_ref[0] + degp_ref[1]
  nrm = lax.rsqrt(jnp.maximum(deg, 1.0))
  norm_ref[...] = nrm
  u_ref[0] = x_ref[...] * nrm


def _t1_call(x_pad, degp3):
  return pl.pallas_call(
      _t1_body,
      grid=(NBLK,),
      in_specs=[
          pl.BlockSpec((BR, IN_F), lambda i: (i, 0)),
          pl.BlockSpec((2, BR, 1), lambda i: (0, i, 0)),
      ],
      out_specs=[
          pl.BlockSpec((BR, 1), lambda i: (i, 0)),
          pl.BlockSpec((1, BR, LW), lambda i: (0, i, 0)),
      ],
      out_shape=[
          jax.ShapeDtypeStruct((N_PAD, 1), jnp.float32),
          jax.ShapeDtypeStruct((1, N_PAD, LW), jnp.float32),
      ],
  )(x_pad, degp3)


def _t2_body(p_ref, norm_ref, tx_ref, u2_ref):
  nrm = norm_ref[...]
  t = -(nrm * (p_ref[0, 0] + p_ref[0, 1]))
  tx_ref[...] = t
  u2_ref[0] = nrm * t


def _t2_call(p, norm):
  return pl.pallas_call(
      _t2_body,
      grid=(NBLK,),
      in_specs=[
          pl.BlockSpec((1, 2, BR, LW), lambda i: (0, 0, i, 0)),
          pl.BlockSpec((BR, 1), lambda i: (i, 0)),
      ],
      out_specs=[
          pl.BlockSpec((BR, LW), lambda i: (i, 0)),
          pl.BlockSpec((1, BR, LW), lambda i: (0, i, 0)),
      ],
      out_shape=[
          jax.ShapeDtypeStruct((N_PAD, LW), jnp.float32),
          jax.ShapeDtypeStruct((1, N_PAD, LW), jnp.float32),
      ],
  )(p, norm)


def _t2w_body(p_ref, norm_ref, tx_ref, u2_ref):
  nrm = norm_ref[...]
  t = -(nrm * (p_ref[0] + p_ref[1]))
  tx_ref[...] = t
  u2_ref[...] = nrm * t


def _t2w_call(p64, norm):
  return pl.pallas_call(
      _t2w_body,
      grid=(NBLK,),
      in_specs=[
          pl.BlockSpec((2, BR, HID), lambda i: (0, i, 0)),
          pl.BlockSpec((BR, 1), lambda i: (i, 0)),
      ],
      out_specs=[
          pl.BlockSpec((BR, HID), lambda i: (i, 0)),
          pl.BlockSpec((BR, HID), lambda i: (i, 0)),
      ],
      out_shape=[
          jax.ShapeDtypeStruct((N_PAD, HID), jnp.float32),
          jax.ShapeDtypeStruct((N_PAD, HID), jnp.float32),
      ],
  )(p64, norm)


def _cheb_wide(h_ref, tx_ref, m, w_ref, b_ref):
  z = jnp.dot(h_ref[...], w_ref[0] - w_ref[2],
              preferred_element_type=jnp.float32)
  z += jnp.dot(tx_ref[...], w_ref[1], preferred_element_type=jnp.float32)
  z -= 2.0 * jnp.dot(m, w_ref[2], preferred_element_type=jnp.float32)
  return jnp.maximum(z + b_ref[...], 0.0)


def _t3a_body(h_ref, tx_ref, p2_ref, norm_ref, w_ref, b_ref,
              h_out_ref, u_out_ref):
  nrm = norm_ref[...]
  m = nrm * (p2_ref[0, 0] + p2_ref[0, 1])
  hn = _cheb_wide(h_ref, tx_ref, m, w_ref, b_ref)
  h_out_ref[...] = hn
  u_out_ref[...] = nrm * hn


def _t3a_call(h, tx, p2, norm, w, b2d):
  return pl.pallas_call(
      _t3a_body,
      grid=(NBLK,),
      in_specs=[
          pl.BlockSpec((BR, IN_F), lambda i: (i, 0)),
          pl.BlockSpec((BR, IN_F), lambda i: (i, 0)),
          pl.BlockSpec((1, 2, BR, LW), lambda i: (0, 0, i, 0)),
          pl.BlockSpec((BR, 1), lambda i: (i, 0)),
          pl.BlockSpec((3, IN_F, HID), lambda i: (0, 0, 0)),
          pl.BlockSpec((1, HID), lambda i: (0, 0)),
      ],
      out_specs=[
          pl.BlockSpec((BR, HID), lambda i: (i, 0)),
          pl.BlockSpec((BR, HID), lambda i: (i, 0)),
      ],
      out_shape=[
          jax.ShapeDtypeStruct((N_PAD, HID), jnp.float32),
          jax.ShapeDtypeStruct((N_PAD, HID), jnp.float32),
      ],
  )(h, tx, p2, norm, w, b2d)


def _t3w_body(h_ref, tx_ref, p2_ref, norm_ref, w_ref, b_ref,
              h_out_ref, u_out_ref):
  nrm = norm_ref[...]
  m = nrm * (p2_ref[0] + p2_ref[1])
  hn = _cheb_wide(h_ref, tx_ref, m, w_ref, b_ref)
  h_out_ref[...] = hn
  u_out_ref[...] = nrm * hn


def _t3w_call(h, tx, p264, norm, w, b2d):
  return pl.pallas_call(
      _t3w_body,
      grid=(NBLK,),
      in_specs=[
          pl.BlockSpec((BR, HID), lambda i: (i, 0)),
          pl.BlockSpec((BR, HID), lambda i: (i, 0)),
          pl.BlockSpec((2, BR, HID), lambda i: (0, i, 0)),
          pl.BlockSpec((BR, 1), lambda i: (i, 0)),
          pl.BlockSpec((3, HID, HID), lambda i: (0, 0, 0)),
          pl.BlockSpec((1, HID), lambda i: (0, 0)),
      ],
      out_specs=[
          pl.BlockSpec((BR, HID), lambda i: (i, 0)),
          pl.BlockSpec((BR, HID), lambda i: (i, 0)),
      ],
      out_shape=[
          jax.ShapeDtypeStruct((N_PAD, HID), jnp.float32),
          jax.ShapeDtypeStruct((N_PAD, HID), jnp.float32),
      ],
  )(h, tx, p264, norm, w, b2d)


def _t3f_body(h_ref, tx_ref, p2_ref, norm_ref, w_ref, b_ref,
              fcw_ref, fcb_ref, hsum_ref, logits_ref):
  i = pl.program_id(0)
  nrm = norm_ref[...]
  m = nrm * (p2_ref[0] + p2_ref[1])
  hn = _cheb_wide(h_ref, tx_ref, m, w_ref, b_ref)
  row = lax.broadcasted_iota(jnp.int32, (BR, 1), 0) + i * BR
  hn = jnp.where(row < N, hn, 0.0)

  @pl.when(i == 0)
  def _():
    hsum_ref[...] = jnp.zeros_like(hsum_ref)

  hsum_ref[...] += jnp.sum(hn, axis=0, keepdims=True)

  @pl.when(i == NBLK - 1)
  def _():
    hg = hsum_ref[...] * (1.0 / N)
    logits_ref[...] = jnp.dot(hg, fcw_ref[...],
                              preferred_element_type=jnp.float32) + fcb_ref[...]


def _t3f_call(h, tx, p264, norm, w, b2d, fc_w, fcb2d):
  _, logits = pl.pallas_call(
      _t3f_body,
      grid=(NBLK,),
      in_specs=[
          pl.BlockSpec((BR, HID), lambda i: (i, 0)),
          pl.BlockSpec((BR, HID), lambda i: (i, 0)),
          pl.BlockSpec((2, BR, HID), lambda i: (0, i, 0)),
          pl.BlockSpec((BR, 1), lambda i: (i, 0)),
          pl.BlockSpec((3, HID, HID), lambda i: (0, 0, 0)),
          pl.BlockSpec((1, HID), lambda i: (0, 0)),
          pl.BlockSpec((HID, OUT_F), lambda i: (0, 0)),
          pl.BlockSpec((1, OUT_F), lambda i: (0, 0)),
      ],
      out_specs=[
          pl.BlockSpec((1, HID), lambda i: (0, 0)),
          pl.BlockSpec((1, OUT_F), lambda i: (0, 0)),
      ],
      out_shape=[
          jax.ShapeDtypeStruct((1, HID), jnp.float32),
          jax.ShapeDtypeStruct((1, OUT_F), jnp.float32),
      ],
  )(h, tx, p264, norm, w, b2d, fc_w, fcb2d)
  return logits


@jax.jit
def kernel(x, edge_index, W1, b1, W2, b2, W3, b3, fc_w, fc_b):
  src = edge_index[0]
  dst = edge_index[1]
  pad_ids = lax.iota(jnp.int32, PAD)
  src_p = jnp.concatenate([src, pad_ids % 512]).reshape(E_PAD // SUB, SUB)
  dst_p = jnp.concatenate([dst, N + pad_ids % DUMP]).reshape(E_PAD // SUB, SUB)

  degp = _deg_kernel(dst_p)
  degp3 = degp.reshape(NC, N_PAD, 1)
  x_pad = jnp.pad(x, ((0, N_PAD - N), (0, 0)))
  norm, u1 = _t1_call(x_pad, degp3)

  p1 = _scatter_c1(src_p, dst_p, u1)
  tx1, u2 = _t2_call(p1, norm)
  p2 = _scatter_c1(src_p, dst_p, u2)
  h, u64 = _t3a_call(x_pad, tx1, p2, norm, W1, b1.reshape(1, HID))

  for layer in (1, 2):
    p1 = _scatter_c4(src_p, dst_p, u64.reshape(4 * N_PAD, LW))
    tx, u2_64 = _t2w_call(p1.reshape(NC, N_PAD, HID), norm)
    p2 = _scatter_c4(src_p, dst_p, u2_64.reshape(4 * N_PAD, LW))
    p264 = p2.reshape(NC, N_PAD, HID)
    if layer == 1:
      h, u64 = _t3w_call(h, tx, p264, norm, W2, b2.reshape(1, HID))
    else:
      logits = _t3f_call(h, tx, p264, norm, W3, b3.reshape(1, HID),
                         fc_w, fc_b.reshape(1, OUT_F))
  return logits


# speedup vs baseline: 1.2236x; 1.2236x over previous
import functools

import jax
import jax.numpy as jnp
from jax import lax
from jax.experimental import pallas as pl
from jax.experimental.pallas import tpu as pltpu
from jax.experimental.pallas import tpu_sc as plsc

N = 100000
E = 3200000
IN_F = 16
HID = 64
OUT_F = 8

NC = 2
NS = 16
LW = 16
SUB = 128
WIN_ROWS = 5
WIN = SUB * WIN_ROWS
N_WIN = 160
EDGES_PER_TILE = WIN * N_WIN
E_PAD = EDGES_PER_TILE * NC * NS
PAD = E_PAD - E
DUMP = 352
N_PAD = N + DUMP
ROWS_PER_TILE_IDX = EDGES_PER_TILE // SUB
SLICE = N_PAD // NS

BR = 2048
NBLK = N_PAD // BR
PKR = BR * LW // 128
NPK = N_PAD * LW // 128
DGR = BR // 128
NDG = N_PAD // 128

SEGS = [(k * WIN, WIN) for k in range(SLICE // WIN)]
if SLICE % WIN:
  SEGS.append((SLICE // WIN * WIN, SLICE % WIN))

Q = 26624
QA = Q + 128
QF = Q // NS
CAPR = 3456
RB = 4


def _scatter_sc_body(n_chunks, src_hbm, dst_hbm, u_hbm, out_hbm,
                     src_v0, src_v1, dst_v0, dst_v1, rows_v0, rows_v1, agg_sh,
                     sem_g0, sem_g1, sem_s0, sem_s1):
  c = lax.axis_index("c")
  s = lax.axis_index("s")
  row_base = (c * NS + s) * ROWS_PER_TILE_IDX
  src_v = (src_v0, src_v1)
  dst_v = (dst_v0, dst_v1)
  rows_v = (rows_v0, rows_v1)
  sem_g = (sem_g0, sem_g1)
  sem_s = (sem_s0, sem_s1)

  def _gathers(ch, b):
    return [pltpu.make_async_copy(u_hbm.at[ch].at[src_v[b].at[j]],
                                  rows_v[b].at[pl.ds(j * SUB, SUB)], sem_g[b])
            for j in range(WIN_ROWS)]

  def _scatters(b):
    return [pltpu.make_async_copy(rows_v[b].at[pl.ds(j * SUB, SUB)],
                                  agg_sh.at[dst_v[b].at[j]], sem_s[b])
            for j in range(WIN_ROWS)]

  def _load_idx(b, w):
    r0 = row_base + w * WIN_ROWS
    pltpu.sync_copy(src_hbm.at[pl.ds(r0, WIN_ROWS)], src_v[b])
    pltpu.sync_copy(dst_hbm.at[pl.ds(r0, WIN_ROWS)], dst_v[b])

  for ch in range(n_chunks):
    def _z(i, _):
      rows_v0[i, :] = jnp.zeros((LW,), jnp.float32)
      return 0
    lax.fori_loop(0, WIN, _z, 0)
    zb = s * SLICE
    for off, sz in SEGS:
      pltpu.sync_copy(rows_v0.at[pl.ds(0, sz)], agg_sh.at[pl.ds(zb + off, sz)])
    plsc.subcore_barrier()

    for b in range(2):
      _load_idx(b, b)
      for cp in _gathers(ch, b):
        cp.start()

    def _pair(k, _):
      for b in range(2):
        for cp in _gathers(ch, b):
          cp.wait()
        for cp in _scatters(b):
          cp.start(add=True)
      for b in range(2):
        w = 2 * k + 2 + b

        @pl.when(w < N_WIN)
        def _():
          for cp in _scatters(b):
            cp.wait()
          _load_idx(b, w)
          for cp in _gathers(ch, b):
            cp.start()
      return 0
    lax.fori_loop(0, N_WIN // 2, _pair, 0)
    for b in range(2):
      for cp in _scatters(b):
        cp.wait()
    plsc.subcore_barrier()

    fb = s * SLICE
    for off, sz in SEGS:
      pltpu.sync_copy(agg_sh.at[pl.ds(fb + off, sz)],
                      out_hbm.at[ch, c, pl.ds(fb + off, sz)])


_scatter_c1 = pl.kernel(
    functools.partial(_scatter_sc_body, 1),
    out_type=jax.ShapeDtypeStruct((1, NC, N_PAD, LW), jnp.float32),
    mesh=plsc.VectorSubcoreMesh(core_axis_name="c", subcore_axis_name="s"),
    scratch_types=[
        pltpu.VMEM((WIN_ROWS, SUB), jnp.int32),
        pltpu.VMEM((WIN_ROWS, SUB), jnp.int32),
        pltpu.VMEM((WIN_ROWS, SUB), jnp.int32),
        pltpu.VMEM((WIN_ROWS, SUB), jnp.int32),
        pltpu.VMEM((WIN, LW), jnp.float32),
        pltpu.VMEM((WIN, LW), jnp.float32),
        pltpu.VMEM_SHARED((N_PAD, LW), jnp.float32),
        pltpu.SemaphoreType.DMA,
        pltpu.SemaphoreType.DMA,
        pltpu.SemaphoreType.DMA,
        pltpu.SemaphoreType.DMA,
    ],
    compiler_params=pltpu.CompilerParams(use_tc_tiling_on_sc=False, needs_layout_passes=False),
)


def _part_sc_body(src_hbm, dst_hbm, qsrc_hbm, qdst_hbm, qcnt_hbm,
                  in_src, in_dst, ss0, ss1, ss2, ss3, ds0, ds1, ds2, ds3,
                  cnt_v, cnt_sm):
  c = lax.axis_index("c")
  s = lax.axis_index("s")
  row_base = (c * NS + s) * ROWS_PER_TILE_IDX
  sstg = (ss0, ss1, ss2, ss3)
  dstg = (ds0, ds1, ds2, ds3)

  @pl.when(s == 0)
  def _():
    for q in range(4):
      cnt_sm[q] = 0
  plsc.subcore_barrier()

  lanes = lax.iota(jnp.int32, LW)

  def _flush(qt):
    rown = plsc.fetch_and_add(cnt_sm.at[qt], 1, subcore_id=0)
    pltpu.sync_copy(sstg[qt].at[pl.ds(0, SUB)], qsrc_hbm.at[c, qt, rown])
    pltpu.sync_copy(dstg[qt].at[pl.ds(0, SUB)], qdst_hbm.at[c, qt, rown])
    for j in range(SUB // LW):
      sstg[qt][pl.ds(j * LW, LW)] = sstg[qt][pl.ds(SUB + j * LW, LW)]
      dstg[qt][pl.ds(j * LW, LW)] = dstg[qt][pl.ds(SUB + j * LW, LW)]

  def _step(k, ps):
    r0 = row_base + k * RB
    pltpu.sync_copy(src_hbm.at[pl.ds(r0, RB)], in_src)
    pltpu.sync_copy(dst_hbm.at[pl.ds(r0, RB)], in_dst)
    for r in range(RB):
      for j in range(SUB // LW):
        sl = (r, pl.ds(j * LW, LW))
        v = in_src[sl]
        d = in_dst[sl]
        qv = ((d >= Q).astype(jnp.int32) + (d >= 2 * Q).astype(jnp.int32)
              + (d >= 3 * Q).astype(jnp.int32))
        new_ps = []
        for qt in range(4):
          m = qv == qt
          cnt = plsc.all_reduce_population_count(m)[0]
          plsc.store_compressed(sstg[qt].at[pl.ds(ps[qt], LW)], v, mask=m)
          plsc.store_compressed(dstg[qt].at[pl.ds(ps[qt], LW)], d - qt * Q,
                                mask=m)
          new_ps.append(ps[qt] + cnt)
        ps = tuple(new_ps)
      new_ps = []
      for qt in range(4):
        full = ps[qt] >= SUB

        @pl.when(full)
        def _():
          _flush(qt)
        new_ps.append(ps[qt] - SUB * full.astype(jnp.int32))
      ps = tuple(new_ps)
    return ps

  ps = lax.fori_loop(0, ROWS_PER_TILE_IDX // RB, _step,
                     tuple(jnp.int32(0) for _ in range(4)))

  for qt in range(4):
    for j in range(SUB // LW):
      pos = ps[qt] + j * LW
      sstg[qt][pl.ds(pos, LW)] = lanes + j * LW
      dstg[qt][pl.ds(pos, LW)] = Q + lanes + j * LW
    _flush(qt)

  plsc.subcore_barrier()

  @pl.when(s == 0)
  def _():
    out = jnp.zeros((LW,), jnp.int32)
    for q in range(4):
      out = jnp.where(lanes == q, cnt_sm[q], out)
    cnt_v[...] = out
    pltpu.sync_copy(cnt_v, qcnt_hbm.at[c])


_part_kernel = pl.kernel(
    _part_sc_body,
    out_type=(
        jax.ShapeDtypeStruct((NC, 4, CAPR, SUB), jnp.int32),
        jax.ShapeDtypeStruct((NC, 4, CAPR, SUB), jnp.int32),
        jax.ShapeDtypeStruct((NC, LW), jnp.int32),
    ),
    mesh=plsc.VectorSubcoreMesh(core_axis_name="c", subcore_axis_name="s"),
    scratch_types=[
        pltpu.VMEM((RB, SUB), jnp.int32),
        pltpu.VMEM((RB, SUB), jnp.int32),
    ] + [pltpu.VMEM((3 * SUB,), jnp.int32) for _ in range(8)] + [
        pltpu.VMEM((LW,), jnp.int32),
        pltpu.SMEM((4,), jnp.int32),
    ],
    compiler_params=pltpu.CompilerParams(use_tc_tiling_on_sc=False, needs_layout_passes=False),
)


def _scat64q_body(qsrc_hbm, qdst_hbm, qcnt_hbm, u_hbm, out_hbm,
                  src_v0, src_v1, dst_v0, dst_v1, rows_v0, rows_v1,
                  cnt_v, agg_sh, sem_g0, sem_g1, sem_s0, sem_s1):
  c = lax.axis_index("c")
  s = lax.axis_index("s")
  src_v = (src_v0, src_v1)
  dst_v = (dst_v0, dst_v1)
  rows_v = (rows_v0, rows_v1)
  sem_g = (sem_g0, sem_g1)
  sem_s = (sem_s0, sem_s1)

  pltpu.sync_copy(qcnt_hbm.at[c], cnt_v)
  cv = cnt_v[...]

  def _gather(b):
    return pltpu.make_async_copy(u_hbm.at[src_v[b].at[0]], rows_v[b], sem_g[b])

  def _scatter(b):
    return pltpu.make_async_copy(rows_v[b], agg_sh.at[dst_v[b].at[0]],
                                 sem_s[b])

  for q in range(4):
    nrows = cv[q]

    def _z(i, _):
      for j in range(HID // LW):
        rows_v0[i, pl.ds(j * LW, LW)] = jnp.zeros((LW,), jnp.float32)
      return 0
    lax.fori_loop(0, SUB, _z, 0)
    zb = s * QF
    for k in range(QF // SUB):
      pltpu.sync_copy(rows_v0, agg_sh.at[pl.ds(zb + k * SUB, SUB)])
    plsc.subcore_barrier()

    nb = (nrows - s + NS - 1) // NS

    def _load(b, w):
      rr = s + w * NS
      pltpu.sync_copy(qsrc_hbm.at[c, q, pl.ds(rr, 1)], src_v[b])
      pltpu.sync_copy(qdst_hbm.at[c, q, pl.ds(rr, 1)], dst_v[b])

    for b in range(2):
      @pl.when(nb > b)
      def _():
        _load(b, b)
        _gather(b).start()

    def _pair(k, _):
      for b in range(2):
        w = 2 * k + b

        @pl.when(w < nb)
        def _():
          _gather(b).wait()
          _scatter(b).start(add=True)
      for b in range(2):
        w2 = 2 * k + 2 + b

        @pl.when(w2 < nb)
        def _():
          _scatter(b).wait()
          _load(b, w2)
          _gather(b).start()
      return 0
    lax.fori_loop(0, (nb + 1) // 2, _pair, 0)
    for b in range(2):
      @pl.when(nb > b)
      def _():
        _scatter(b).wait()
    plsc.subcore_barrier()

    pltpu.sync_copy(agg_sh.at[pl.ds(s * QF, QF)],
                    out_hbm.at[c, q, pl.ds(s * QF, QF)])


_scat64q = pl.kernel(
    _scat64q_body,
    out_type=jax.ShapeDtypeStruct((NC, 4, Q, HID), jnp.float32),
    mesh=plsc.VectorSubcoreMesh(core_axis_name="c", subcore_axis_name="s"),
    scratch_types=[
        pltpu.VMEM((1, SUB), jnp.int32),
        pltpu.VMEM((1, SUB), jnp.int32),
        pltpu.VMEM((1, SUB), jnp.int32),
        pltpu.VMEM((1, SUB), jnp.int32),
        pltpu.VMEM((SUB, HID), jnp.float32),
        pltpu.VMEM((SUB, HID), jnp.float32),
        pltpu.VMEM((LW,), jnp.int32),
        pltpu.VMEM_SHARED((QA, HID), jnp.float32),
        pltpu.SemaphoreType.DMA,
        pltpu.SemaphoreType.DMA,
        pltpu.SemaphoreType.DMA,
        pltpu.SemaphoreType.DMA,
    ],
    compiler_params=pltpu.CompilerParams(use_tc_tiling_on_sc=False, needs_layout_passes=False),
)


def _deg_sc_body(dst_hbm, out_hbm, dst_v0, dst_v1, ones_v, zeros_v, deg_sh,
                 sem_s0, sem_s1):
  c = lax.axis_index("c")
  s = lax.axis_index("s")
  row_base = (c * NS + s) * ROWS_PER_TILE_IDX
  dst_v = (dst_v0, dst_v1)
  sem_s = (sem_s0, sem_s1)

  def _o(i, _):
    ones_v[pl.ds(i * LW, LW)] = jnp.ones((LW,), jnp.float32)
    return 0
  lax.fori_loop(0, SUB // LW, _o, 0)

  def _z(i, _):
    zeros_v[pl.ds(i * LW, LW)] = jnp.zeros((LW,), jnp.float32)
    return 0
  lax.fori_loop(0, WIN // LW, _z, 0)

  zb = s * SLICE
  for off, sz in SEGS:
    pltpu.sync_copy(zeros_v.at[pl.ds(0, sz)], deg_sh.at[pl.ds(zb + off, sz)])
  plsc.subcore_barrier()

  def _scatters(b):
    return [pltpu.make_async_copy(ones_v, deg_sh.at[dst_v[b].at[j]], sem_s[b])
            for j in range(WIN_ROWS)]

  def _load_idx(b, w):
    r0 = row_base + w * WIN_ROWS
    pltpu.sync_copy(dst_hbm.at[pl.ds(r0, WIN_ROWS)], dst_v[b])

  for b in range(2):
    _load_idx(b, b)

  def _pair(k, _):
    for b in range(2):
      for cp in _scatters(b):
        cp.start(add=True)
    for b in range(2):
      w = 2 * k + 2 + b

      @pl.when(w < N_WIN)
      def _():
        for cp in _scatters(b):
          cp.wait()
        _load_idx(b, w)
    return 0
  lax.fori_loop(0, N_WIN // 2, _pair, 0)
  for b in range(2):
    for cp in _scatters(b):
      cp.wait()
  plsc.subcore_barrier()

  fb = s * SLICE
  for off, sz in SEGS:
    pltpu.sync_copy(deg_sh.at[pl.ds(fb + off, sz)],
                    out_hbm.at[c, 0, pl.ds(fb + off, sz)])


_deg_kernel = pl.kernel(
    _deg_sc_body,
    out_type=jax.ShapeDtypeStruct((NC, 1, N_PAD), jnp.float32),
    mesh=plsc.VectorSubcoreMesh(core_axis_name="c", subcore_axis_name="s"),
    scratch_types=[
        pltpu.VMEM((WIN_ROWS, SUB), jnp.int32),
        pltpu.VMEM((WIN_ROWS, SUB), jnp.int32),
        pltpu.VMEM((SUB,), jnp.float32),
        pltpu.VMEM((WIN,), jnp.float32),
        pltpu.VMEM_SHARED((N_PAD,), jnp.float32),
        pltpu.SemaphoreType.DMA,
        pltpu.SemaphoreType.DMA,
    ],
    compiler_params=pltpu.CompilerParams(use_tc_tiling_on_sc=False, needs_layout_passes=False),
)


def _t1_body(x_ref, degp_ref, norm_ref, u_ref):
  deg = degp_ref[0] + degp_ref[1]
  nrm = lax.rsqrt(jnp.maximum(deg, 1.0))
  norm_ref[...] = nrm
  u_ref[0] = x_ref[...] * nrm


def _t1_call(x_pad, degp3):
  return pl.pallas_call(
      _t1_body,
      grid=(NBLK,),
      in_specs=[
          pl.BlockSpec((BR, IN_F), lambda i: (i, 0)),
          pl.BlockSpec((2, BR, 1), lambda i: (0, i, 0)),
      ],
      out_specs=[
          pl.BlockSpec((BR, 1), lambda i: (i, 0)),
          pl.BlockSpec((1, BR, LW), lambda i: (0, i, 0)),
      ],
      out_shape=[
          jax.ShapeDtypeStruct((N_PAD, 1), jnp.float32),
          jax.ShapeDtypeStruct((1, N_PAD, LW), jnp.float32),
      ],
  )(x_pad, degp3)


def _t2_body(p_ref, norm_ref, tx_ref, u2_ref):
  nrm = norm_ref[...]
  t = -(nrm * (p_ref[0, 0] + p_ref[0, 1]))
  tx_ref[...] = t
  u2_ref[0] = nrm * t


def _t2_call(p, norm):
  return pl.pallas_call(
      _t2_body,
      grid=(NBLK,),
      in_specs=[
          pl.BlockSpec((1, 2, BR, LW), lambda i: (0, 0, i, 0)),
          pl.BlockSpec((BR, 1), lambda i: (i, 0)),
      ],
      out_specs=[
          pl.BlockSpec((BR, LW), lambda i: (i, 0)),
          pl.BlockSpec((1, BR, LW), lambda i: (0, i, 0)),
      ],
      out_shape=[
          jax.ShapeDtypeStruct((N_PAD, LW), jnp.float32),
          jax.ShapeDtypeStruct((1, N_PAD, LW), jnp.float32),
      ],
  )(p, norm)


def _t2w_body(p_ref, norm_ref, tx_ref, u2_ref):
  nrm = norm_ref[...]
  t = -(nrm * (p_ref[0] + p_ref[1]))
  tx_ref[...] = t
  u2_ref[...] = nrm * t


def _t2w_call(p64, norm):
  return pl.pallas_call(
      _t2w_body,
      grid=(NBLK,),
      in_specs=[
          pl.BlockSpec((2, BR, HID), lambda i: (0, i, 0)),
          pl.BlockSpec((BR, 1), lambda i: (i, 0)),
      ],
      out_specs=[
          pl.BlockSpec((BR, HID), lambda i: (i, 0)),
          pl.BlockSpec((BR, HID), lambda i: (i, 0)),
      ],
      out_shape=[
          jax.ShapeDtypeStruct((N_PAD, HID), jnp.float32),
          jax.ShapeDtypeStruct((N_PAD, HID), jnp.float32),
      ],
  )(p64, norm)


def _cheb_wide(h_ref, tx_ref, m, w_ref, b_ref):
  z = jnp.dot(h_ref[...], w_ref[0] - w_ref[2],
              preferred_element_type=jnp.float32)
  z += jnp.dot(tx_ref[...], w_ref[1], preferred_element_type=jnp.float32)
  z -= 2.0 * jnp.dot(m, w_ref[2], preferred_element_type=jnp.float32)
  return jnp.maximum(z + b_ref[...], 0.0)


def _t3a_body(h_ref, tx_ref, p2_ref, norm_ref, w_ref, b_ref,
              h_out_ref, u_out_ref):
  nrm = norm_ref[...]
  m = nrm * (p2_ref[0, 0] + p2_ref[0, 1])
  hn = _cheb_wide(h_ref, tx_ref, m, w_ref, b_ref)
  h_out_ref[...] = hn
  u_out_ref[...] = nrm * hn


def _t3a_call(h, tx, p2, norm, w, b2d):
  return pl.pallas_call(
      _t3a_body,
      grid=(NBLK,),
      in_specs=[
          pl.BlockSpec((BR, IN_F), lambda i: (i, 0)),
          pl.BlockSpec((BR, IN_F), lambda i: (i, 0)),
          pl.BlockSpec((1, 2, BR, LW), lambda i: (0, 0, i, 0)),
          pl.BlockSpec((BR, 1), lambda i: (i, 0)),
          pl.BlockSpec((3, IN_F, HID), lambda i: (0, 0, 0)),
          pl.BlockSpec((1, HID), lambda i: (0, 0)),
      ],
      out_specs=[
          pl.BlockSpec((BR, HID), lambda i: (i, 0)),
          pl.BlockSpec((BR, HID), lambda i: (i, 0)),
      ],
      out_shape=[
          jax.ShapeDtypeStruct((N_PAD, HID), jnp.float32),
          jax.ShapeDtypeStruct((N_PAD, HID), jnp.float32),
      ],
  )(h, tx, p2, norm, w, b2d)


def _t3w_body(h_ref, tx_ref, p2_ref, norm_ref, w_ref, b_ref,
              h_out_ref, u_out_ref):
  nrm = norm_ref[...]
  m = nrm * (p2_ref[0] + p2_ref[1])
  hn = _cheb_wide(h_ref, tx_ref, m, w_ref, b_ref)
  h_out_ref[...] = hn
  u_out_ref[...] = nrm * hn


def _t3w_call(h, tx, p264, norm, w, b2d):
  return pl.pallas_call(
      _t3w_body,
      grid=(NBLK,),
      in_specs=[
          pl.BlockSpec((BR, HID), lambda i: (i, 0)),
          pl.BlockSpec((BR, HID), lambda i: (i, 0)),
          pl.BlockSpec((2, BR, HID), lambda i: (0, i, 0)),
          pl.BlockSpec((BR, 1), lambda i: (i, 0)),
          pl.BlockSpec((3, HID, HID), lambda i: (0, 0, 0)),
          pl.BlockSpec((1, HID), lambda i: (0, 0)),
      ],
      out_specs=[
          pl.BlockSpec((BR, HID), lambda i: (i, 0)),
          pl.BlockSpec((BR, HID), lambda i: (i, 0)),
      ],
      out_shape=[
          jax.ShapeDtypeStruct((N_PAD, HID), jnp.float32),
          jax.ShapeDtypeStruct((N_PAD, HID), jnp.float32),
      ],
  )(h, tx, p264, norm, w, b2d)


def _t3f_body(h_ref, tx_ref, p2_ref, norm_ref, w_ref, b_ref,
              fcw_ref, fcb_ref, hsum_ref, logits_ref):
  i = pl.program_id(0)
  nrm = norm_ref[...]
  m = nrm * (p2_ref[0] + p2_ref[1])
  hn = _cheb_wide(h_ref, tx_ref, m, w_ref, b_ref)
  row = lax.broadcasted_iota(jnp.int32, (BR, 1), 0) + i * BR
  hn = jnp.where(row < N, hn, 0.0)

  @pl.when(i == 0)
  def _():
    hsum_ref[...] = jnp.zeros_like(hsum_ref)

  hsum_ref[...] += jnp.sum(hn, axis=0, keepdims=True)

  @pl.when(i == NBLK - 1)
  def _():
    hg = hsum_ref[...] * (1.0 / N)
    logits_ref[...] = jnp.dot(hg, fcw_ref[...],
                              preferred_element_type=jnp.float32) + fcb_ref[...]


def _t3f_call(h, tx, p264, norm, w, b2d, fc_w, fcb2d):
  _, logits = pl.pallas_call(
      _t3f_body,
      grid=(NBLK,),
      in_specs=[
          pl.BlockSpec((BR, HID), lambda i: (i, 0)),
          pl.BlockSpec((BR, HID), lambda i: (i, 0)),
          pl.BlockSpec((2, BR, HID), lambda i: (0, i, 0)),
          pl.BlockSpec((BR, 1), lambda i: (i, 0)),
          pl.BlockSpec((3, HID, HID), lambda i: (0, 0, 0)),
          pl.BlockSpec((1, HID), lambda i: (0, 0)),
          pl.BlockSpec((HID, OUT_F), lambda i: (0, 0)),
          pl.BlockSpec((1, OUT_F), lambda i: (0, 0)),
      ],
      out_specs=[
          pl.BlockSpec((1, HID), lambda i: (0, 0)),
          pl.BlockSpec((1, OUT_F), lambda i: (0, 0)),
      ],
      out_shape=[
          jax.ShapeDtypeStruct((1, HID), jnp.float32),
          jax.ShapeDtypeStruct((1, OUT_F), jnp.float32),
      ],
  )(h, tx, p264, norm, w, b2d, fc_w, fcb2d)
  return logits


@jax.jit
def kernel(x, edge_index, W1, b1, W2, b2, W3, b3, fc_w, fc_b):
  src = edge_index[0]
  dst = edge_index[1]
  pad_ids = lax.iota(jnp.int32, PAD)
  src_p = jnp.concatenate([src, pad_ids % 512]).reshape(E_PAD // SUB, SUB)
  dst_p = jnp.concatenate([dst, N + pad_ids % DUMP]).reshape(E_PAD // SUB, SUB)

  degp = _deg_kernel(dst_p)
  degp3 = degp.reshape(NC, N_PAD, 1)
  x_pad = jnp.pad(x, ((0, N_PAD - N), (0, 0)))
  norm, u1 = _t1_call(x_pad, degp3)

  p1 = _scatter_c1(src_p, dst_p, u1)
  tx1, u2 = _t2_call(p1, norm)
  p2 = _scatter_c1(src_p, dst_p, u2)
  h, u64 = _t3a_call(x_pad, tx1, p2, norm, W1, b1.reshape(1, HID))

  qsrc, qdst, qcnt = _part_kernel(src_p, dst_p)
  for layer in (1, 2):
    p1 = _scat64q(qsrc, qdst, qcnt, u64)
    tx, u2_64 = _t2w_call(p1.reshape(NC, 4 * Q, HID), norm)
    p2 = _scat64q(qsrc, qdst, qcnt, u2_64)
    p264 = p2.reshape(NC, 4 * Q, HID)
    if layer == 1:
      h, u64 = _t3w_call(h, tx, p264, norm, W2, b2.reshape(1, HID))
    else:
      logits = _t3f_call(h, tx, p264, norm, W3, b3.reshape(1, HID),
                         fc_w, fc_b.reshape(1, OUT_F))
  return logits


# speedup vs baseline: 1.2250x; 1.0011x over previous
import functools

import jax
import jax.numpy as jnp
from jax import lax
from jax.experimental import pallas as pl
from jax.experimental.pallas import tpu as pltpu
from jax.experimental.pallas import tpu_sc as plsc

N = 100000
E = 3200000
IN_F = 16
HID = 64
OUT_F = 8

NC = 2
NS = 16
LW = 16
SUB = 128
WIN_ROWS = 5
WIN = SUB * WIN_ROWS
N_WIN = 160
EDGES_PER_TILE = WIN * N_WIN
E_PAD = EDGES_PER_TILE * NC * NS
PAD = E_PAD - E
DUMP = 352
N_PAD = N + DUMP
ROWS_PER_TILE_IDX = EDGES_PER_TILE // SUB
SLICE = N_PAD // NS

BR = 2048
NBLK = N_PAD // BR
PKR = BR * LW // 128
NPK = N_PAD * LW // 128
DGR = BR // 128
NDG = N_PAD // 128

SEGS = [(k * WIN, WIN) for k in range(SLICE // WIN)]
if SLICE % WIN:
  SEGS.append((SLICE // WIN * WIN, SLICE % WIN))

Q = 26624
QA = Q + 128
QF = Q // NS
CAPR = 3456
RB = 4


def _scatter_sc_body(n_chunks, src_hbm, dst_hbm, u_hbm, out_hbm,
                     src_v0, src_v1, dst_v0, dst_v1, rows_v0, rows_v1, agg_sh,
                     sem_g0, sem_g1, sem_s0, sem_s1):
  c = lax.axis_index("c")
  s = lax.axis_index("s")
  row_base = (c * NS + s) * ROWS_PER_TILE_IDX
  src_v = (src_v0, src_v1)
  dst_v = (dst_v0, dst_v1)
  rows_v = (rows_v0, rows_v1)
  sem_g = (sem_g0, sem_g1)
  sem_s = (sem_s0, sem_s1)

  def _gathers(ch, b):
    return [pltpu.make_async_copy(u_hbm.at[ch].at[src_v[b].at[j]],
                                  rows_v[b].at[pl.ds(j * SUB, SUB)], sem_g[b])
            for j in range(WIN_ROWS)]

  def _scatters(b):
    return [pltpu.make_async_copy(rows_v[b].at[pl.ds(j * SUB, SUB)],
                                  agg_sh.at[dst_v[b].at[j]], sem_s[b])
            for j in range(WIN_ROWS)]

  def _load_idx(b, w):
    r0 = row_base + w * WIN_ROWS
    pltpu.sync_copy(src_hbm.at[pl.ds(r0, WIN_ROWS)], src_v[b])
    pltpu.sync_copy(dst_hbm.at[pl.ds(r0, WIN_ROWS)], dst_v[b])

  for ch in range(n_chunks):
    def _z(i, _):
      rows_v0[i, :] = jnp.zeros((LW,), jnp.float32)
      return 0
    lax.fori_loop(0, WIN, _z, 0)
    zb = s * SLICE
    for off, sz in SEGS:
      pltpu.sync_copy(rows_v0.at[pl.ds(0, sz)], agg_sh.at[pl.ds(zb + off, sz)])
    plsc.subcore_barrier()

    for b in range(2):
      _load_idx(b, b)
      for cp in _gathers(ch, b):
        cp.start()

    def _pair(k, _):
      for b in range(2):
        for cp in _gathers(ch, b):
          cp.wait()
        for cp in _scatters(b):
          cp.start(add=True)
      for b in range(2):
        w = 2 * k + 2 + b

        @pl.when(w < N_WIN)
        def _():
          for cp in _scatters(b):
            cp.wait()
          _load_idx(b, w)
          for cp in _gathers(ch, b):
            cp.start()
      return 0
    lax.fori_loop(0, N_WIN // 2, _pair, 0)
    for b in range(2):
      for cp in _scatters(b):
        cp.wait()
    plsc.subcore_barrier()

    fb = s * SLICE
    for off, sz in SEGS:
      pltpu.sync_copy(agg_sh.at[pl.ds(fb + off, sz)],
                      out_hbm.at[ch, c, pl.ds(fb + off, sz)])


_scatter_c1 = pl.kernel(
    functools.partial(_scatter_sc_body, 1),
    out_type=jax.ShapeDtypeStruct((1, NC, N_PAD, LW), jnp.float32),
    mesh=plsc.VectorSubcoreMesh(core_axis_name="c", subcore_axis_name="s"),
    scratch_types=[
        pltpu.VMEM((WIN_ROWS, SUB), jnp.int32),
        pltpu.VMEM((WIN_ROWS, SUB), jnp.int32),
        pltpu.VMEM((WIN_ROWS, SUB), jnp.int32),
        pltpu.VMEM((WIN_ROWS, SUB), jnp.int32),
        pltpu.VMEM((WIN, LW), jnp.float32),
        pltpu.VMEM((WIN, LW), jnp.float32),
        pltpu.VMEM_SHARED((N_PAD, LW), jnp.float32),
        pltpu.SemaphoreType.DMA,
        pltpu.SemaphoreType.DMA,
        pltpu.SemaphoreType.DMA,
        pltpu.SemaphoreType.DMA,
    ],
    compiler_params=pltpu.CompilerParams(use_tc_tiling_on_sc=False, needs_layout_passes=False),
)


def _part_sc_body(src_hbm, dst_hbm, qsrc_hbm, qdst_hbm, qcnt_hbm,
                  in_src, in_dst, ss0, ss1, ss2, ss3, ds0, ds1, ds2, ds3,
                  cnt_v, cnt_sm):
  c = lax.axis_index("c")
  s = lax.axis_index("s")
  row_base = (c * NS + s) * ROWS_PER_TILE_IDX
  sstg = (ss0, ss1, ss2, ss3)
  dstg = (ds0, ds1, ds2, ds3)

  @pl.when(s == 0)
  def _():
    for q in range(4):
      cnt_sm[q] = 0
  plsc.subcore_barrier()

  lanes = lax.iota(jnp.int32, LW)

  def _flush(qt):
    rown = plsc.fetch_and_add(cnt_sm.at[qt], 1, subcore_id=0)
    pltpu.sync_copy(sstg[qt].at[pl.ds(0, SUB)], qsrc_hbm.at[c, qt, rown])
    pltpu.sync_copy(dstg[qt].at[pl.ds(0, SUB)], qdst_hbm.at[c, qt, rown])
    for j in range(SUB // LW):
      sstg[qt][pl.ds(j * LW, LW)] = sstg[qt][pl.ds(SUB + j * LW, LW)]
      dstg[qt][pl.ds(j * LW, LW)] = dstg[qt][pl.ds(SUB + j * LW, LW)]

  def _step(k, ps):
    r0 = row_base + k * RB
    pltpu.sync_copy(src_hbm.at[pl.ds(r0, RB)], in_src)
    pltpu.sync_copy(dst_hbm.at[pl.ds(r0, RB)], in_dst)
    for r in range(RB):
      for j in range(SUB // LW):
        sl = (r, pl.ds(j * LW, LW))
        v = in_src[sl]
        d = in_dst[sl]
        qv = ((d >= Q).astype(jnp.int32) + (d >= 2 * Q).astype(jnp.int32)
              + (d >= 3 * Q).astype(jnp.int32))
        new_ps = []
        for qt in range(4):
          m = qv == qt
          cnt = plsc.all_reduce_population_count(m)[0]
          plsc.store_compressed(sstg[qt].at[pl.ds(ps[qt], LW)], v, mask=m)
          plsc.store_compressed(dstg[qt].at[pl.ds(ps[qt], LW)], d - qt * Q,
                                mask=m)
          new_ps.append(ps[qt] + cnt)
        ps = tuple(new_ps)
      new_ps = []
      for qt in range(4):
        full = ps[qt] >= SUB

        @pl.when(full)
        def _():
          _flush(qt)
        new_ps.append(ps[qt] - SUB * full.astype(jnp.int32))
      ps = tuple(new_ps)
    return ps

  ps = lax.fori_loop(0, ROWS_PER_TILE_IDX // RB, _step,
                     tuple(jnp.int32(0) for _ in range(4)))

  for qt in range(4):
    for j in range(SUB // LW):
      pos = ps[qt] + j * LW
      sstg[qt][pl.ds(pos, LW)] = lanes + j * LW
      dstg[qt][pl.ds(pos, LW)] = Q + lanes + j * LW
    _flush(qt)

  plsc.subcore_barrier()

  @pl.when(s == 0)
  def _():
    out = jnp.zeros((LW,), jnp.int32)
    for q in range(4):
      out = jnp.where(lanes == q, cnt_sm[q], out)
    cnt_v[...] = out
    pltpu.sync_copy(cnt_v, qcnt_hbm.at[c])


_part_kernel = pl.kernel(
    _part_sc_body,
    out_type=(
        jax.ShapeDtypeStruct((NC, 4, CAPR, SUB), jnp.int32),
        jax.ShapeDtypeStruct((NC, 4, CAPR, SUB), jnp.int32),
        jax.ShapeDtypeStruct((NC, LW), jnp.int32),
    ),
    mesh=plsc.VectorSubcoreMesh(core_axis_name="c", subcore_axis_name="s"),
    scratch_types=[
        pltpu.VMEM((RB, SUB), jnp.int32),
        pltpu.VMEM((RB, SUB), jnp.int32),
    ] + [pltpu.VMEM((3 * SUB,), jnp.int32) for _ in range(8)] + [
        pltpu.VMEM((LW,), jnp.int32),
        pltpu.SMEM((4,), jnp.int32),
    ],
    compiler_params=pltpu.CompilerParams(use_tc_tiling_on_sc=False, needs_layout_passes=False),
)


def _scat64q_body(qsrc_hbm, qdst_hbm, qcnt_hbm, u_hbm, out_hbm,
                  src_v0, src_v1, dst_v0, dst_v1, rows_v0, rows_v1,
                  cnt_v, agg_sh, sem_g0, sem_g1, sem_s0, sem_s1):
  c = lax.axis_index("c")
  s = lax.axis_index("s")
  src_v = (src_v0, src_v1)
  dst_v = (dst_v0, dst_v1)
  rows_v = (rows_v0, rows_v1)
  sem_g = (sem_g0, sem_g1)
  sem_s = (sem_s0, sem_s1)

  pltpu.sync_copy(qcnt_hbm.at[c], cnt_v)
  cv = cnt_v[...]

  def _gather(b):
    return pltpu.make_async_copy(u_hbm.at[src_v[b].at[0]], rows_v[b], sem_g[b])

  def _scatter(b):
    return pltpu.make_async_copy(rows_v[b], agg_sh.at[dst_v[b].at[0]],
                                 sem_s[b])

  for q in range(4):
    nrows = cv[q]

    def _z(i, _):
      for j in range(HID // LW):
        rows_v0[i, pl.ds(j * LW, LW)] = jnp.zeros((LW,), jnp.float32)
      return 0
    lax.fori_loop(0, SUB, _z, 0)
    zb = s * QF
    for k in range(QF // SUB):
      pltpu.sync_copy(rows_v0, agg_sh.at[pl.ds(zb + k * SUB, SUB)])
    plsc.subcore_barrier()

    nb = (nrows - s + NS - 1) // NS

    def _load(b, w):
      rr = s + w * NS
      pltpu.sync_copy(qsrc_hbm.at[c, q, pl.ds(rr, 1)], src_v[b])
      pltpu.sync_copy(qdst_hbm.at[c, q, pl.ds(rr, 1)], dst_v[b])

    for b in range(2):
      @pl.when(nb > b)
      def _():
        _load(b, b)
        _gather(b).start()

    def _pair(k, _):
      for b in range(2):
        w = 2 * k + b

        @pl.when(w < nb)
        def _():
          _gather(b).wait()
          _scatter(b).start(add=True)
      for b in range(2):
        w2 = 2 * k + 2 + b

        @pl.when(w2 < nb)
        def _():
          _scatter(b).wait()
          _load(b, w2)
          _gather(b).start()
      return 0
    lax.fori_loop(0, (nb + 1) // 2, _pair, 0)
    for b in range(2):
      @pl.when(nb > b)
      def _():
        _scatter(b).wait()
    plsc.subcore_barrier()

    pltpu.sync_copy(agg_sh.at[pl.ds(s * QF, QF)],
                    out_hbm.at[c, pl.ds(q * Q + s * QF, QF)])


_scat64q = pl.kernel(
    _scat64q_body,
    out_type=jax.ShapeDtypeStruct((NC, 4 * Q, HID), jnp.float32),
    mesh=plsc.VectorSubcoreMesh(core_axis_name="c", subcore_axis_name="s"),
    scratch_types=[
        pltpu.VMEM((1, SUB), jnp.int32),
        pltpu.VMEM((1, SUB), jnp.int32),
        pltpu.VMEM((1, SUB), jnp.int32),
        pltpu.VMEM((1, SUB), jnp.int32),
        pltpu.VMEM((SUB, HID), jnp.float32),
        pltpu.VMEM((SUB, HID), jnp.float32),
        pltpu.VMEM((LW,), jnp.int32),
        pltpu.VMEM_SHARED((QA, HID), jnp.float32),
        pltpu.SemaphoreType.DMA,
        pltpu.SemaphoreType.DMA,
        pltpu.SemaphoreType.DMA,
        pltpu.SemaphoreType.DMA,
    ],
    compiler_params=pltpu.CompilerParams(use_tc_tiling_on_sc=False, needs_layout_passes=False),
)


def _deg_sc_body(dst_hbm, out_hbm, dst_v0, dst_v1, ones_v, zeros_v, deg_sh,
                 sem_s0, sem_s1):
  c = lax.axis_index("c")
  s = lax.axis_index("s")
  row_base = (c * NS + s) * ROWS_PER_TILE_IDX
  dst_v = (dst_v0, dst_v1)
  sem_s = (sem_s0, sem_s1)

  def _o(i, _):
    ones_v[pl.ds(i * LW, LW)] = jnp.ones((LW,), jnp.float32)
    return 0
  lax.fori_loop(0, SUB // LW, _o, 0)

  def _z(i, _):
    zeros_v[pl.ds(i * LW, LW)] = jnp.zeros((LW,), jnp.float32)
    return 0
  lax.fori_loop(0, WIN // LW, _z, 0)

  zb = s * SLICE
  for off, sz in SEGS:
    pltpu.sync_copy(zeros_v.at[pl.ds(0, sz)], deg_sh.at[pl.ds(zb + off, sz)])
  plsc.subcore_barrier()

  def _scatters(b):
    return [pltpu.make_async_copy(ones_v, deg_sh.at[dst_v[b].at[j]], sem_s[b])
            for j in range(WIN_ROWS)]

  def _load_idx(b, w):
    r0 = row_base + w * WIN_ROWS
    pltpu.sync_copy(dst_hbm.at[pl.ds(r0, WIN_ROWS)], dst_v[b])

  for b in range(2):
    _load_idx(b, b)

  def _pair(k, _):
    for b in range(2):
      for cp in _scatters(b):
        cp.start(add=True)
    for b in range(2):
      w = 2 * k + 2 + b

      @pl.when(w < N_WIN)
      def _():
        for cp in _scatters(b):
          cp.wait()
        _load_idx(b, w)
    return 0
  lax.fori_loop(0, N_WIN // 2, _pair, 0)
  for b in range(2):
    for cp in _scatters(b):
      cp.wait()
  plsc.subcore_barrier()

  fb = s * SLICE
  for off, sz in SEGS:
    pltpu.sync_copy(deg_sh.at[pl.ds(fb + off, sz)],
                    out_hbm.at[c, 0, pl.ds(fb + off, sz)])


_deg_kernel = pl.kernel(
    _deg_sc_body,
    out_type=jax.ShapeDtypeStruct((NC, 1, N_PAD), jnp.float32),
    mesh=plsc.VectorSubcoreMesh(core_axis_name="c", subcore_axis_name="s"),
    scratch_types=[
        pltpu.VMEM((WIN_ROWS, SUB), jnp.int32),
        pltpu.VMEM((WIN_ROWS, SUB), jnp.int32),
        pltpu.VMEM((SUB,), jnp.float32),
        pltpu.VMEM((WIN,), jnp.float32),
        pltpu.VMEM_SHARED((N_PAD,), jnp.float32),
        pltpu.SemaphoreType.DMA,
        pltpu.SemaphoreType.DMA,
    ],
    compiler_params=pltpu.CompilerParams(use_tc_tiling_on_sc=False, needs_layout_passes=False),
)


def _t1_body(x_ref, degp_ref, norm_ref, u_ref):
  deg = degp_ref[0] + degp_ref[1]
  nrm = lax.rsqrt(jnp.maximum(deg, 1.0))
  norm_ref[...] = nrm
  u_ref[0] = x_ref[...] * nrm


def _t1_call(x_pad, degp3):
  return pl.pallas_call(
      _t1_body,
      grid=(NBLK,),
      in_specs=[
          pl.BlockSpec((BR, IN_F), lambda i: (i, 0)),
          pl.BlockSpec((2, BR, 1), lambda i: (0, i, 0)),
      ],
      out_specs=[
          pl.BlockSpec((BR, 1), lambda i: (i, 0)),
          pl.BlockSpec((1, BR, LW), lambda i: (0, i, 0)),
      ],
      out_shape=[
          jax.ShapeDtypeStruct((N_PAD, 1), jnp.float32),
          jax.ShapeDtypeStruct((1, N_PAD, LW), jnp.float32),
      ],
  )(x_pad, degp3)


def _t2_body(p_ref, norm_ref, tx_ref, u2_ref):
  nrm = norm_ref[...]
  t = -(nrm * (p_ref[0, 0] + p_ref[0, 1]))
  tx_ref[...] = t
  u2_ref[0] = nrm * t


def _t2_call(p, norm):
  return pl.pallas_call(
      _t2_body,
      grid=(NBLK,),
      in_specs=[
          pl.BlockSpec((1, 2, BR, LW), lambda i: (0, 0, i, 0)),
          pl.BlockSpec((BR, 1), lambda i: (i, 0)),
      ],
      out_specs=[
          pl.BlockSpec((BR, LW), lambda i: (i, 0)),
          pl.BlockSpec((1, BR, LW), lambda i: (0, i, 0)),
      ],
      out_shape=[
          jax.ShapeDtypeStruct((N_PAD, LW), jnp.float32),
          jax.ShapeDtypeStruct((1, N_PAD, LW), jnp.float32),
      ],
  )(p, norm)


def _t2w_body(p_ref, norm_ref, tx_ref, u2_ref):
  nrm = norm_ref[...]
  t = -(nrm * (p_ref[0] + p_ref[1]))
  tx_ref[...] = t
  u2_ref[...] = nrm * t


def _t2w_call(p64, norm):
  return pl.pallas_call(
      _t2w_body,
      grid=(NBLK,),
      in_specs=[
          pl.BlockSpec((2, BR, HID), lambda i: (0, i, 0)),
          pl.BlockSpec((BR, 1), lambda i: (i, 0)),
      ],
      out_specs=[
          pl.BlockSpec((BR, HID), lambda i: (i, 0)),
          pl.BlockSpec((BR, HID), lambda i: (i, 0)),
      ],
      out_shape=[
          jax.ShapeDtypeStruct((N_PAD, HID), jnp.float32),
          jax.ShapeDtypeStruct((N_PAD, HID), jnp.float32),
      ],
  )(p64, norm)


def _cheb_wide(h_ref, tx_ref, m, w_ref, b_ref):
  z = jnp.dot(h_ref[...], w_ref[0] - w_ref[2],
              preferred_element_type=jnp.float32)
  z += jnp.dot(tx_ref[...], w_ref[1], preferred_element_type=jnp.float32)
  z -= 2.0 * jnp.dot(m, w_ref[2], preferred_element_type=jnp.float32)
  return jnp.maximum(z + b_ref[...], 0.0)


def _t3a_body(h_ref, tx_ref, p2_ref, norm_ref, w_ref, b_ref,
              h_out_ref, u_out_ref):
  nrm = norm_ref[...]
  m = nrm * (p2_ref[0, 0] + p2_ref[0, 1])
  hn = _cheb_wide(h_ref, tx_ref, m, w_ref, b_ref)
  h_out_ref[...] = hn
  u_out_ref[...] = nrm * hn


def _t3a_call(h, tx, p2, norm, w, b2d):
  return pl.pallas_call(
      _t3a_body,
      grid=(NBLK,),
      in_specs=[
          pl.BlockSpec((BR, IN_F), lambda i: (i, 0)),
          pl.BlockSpec((BR, IN_F), lambda i: (i, 0)),
          pl.BlockSpec((1, 2, BR, LW), lambda i: (0, 0, i, 0)),
          pl.BlockSpec((BR, 1), lambda i: (i, 0)),
          pl.BlockSpec((3, IN_F, HID), lambda i: (0, 0, 0)),
          pl.BlockSpec((1, HID), lambda i: (0, 0)),
      ],
      out_specs=[
          pl.BlockSpec((BR, HID), lambda i: (i, 0)),
          pl.BlockSpec((BR, HID), lambda i: (i, 0)),
      ],
      out_shape=[
          jax.ShapeDtypeStruct((N_PAD, HID), jnp.float32),
          jax.ShapeDtypeStruct((N_PAD, HID), jnp.float32),
      ],
  )(h, tx, p2, norm, w, b2d)


def _t3w_body(h_ref, tx_ref, p2_ref, norm_ref, w_ref, b_ref,
              h_out_ref, u_out_ref):
  nrm = norm_ref[...]
  m = nrm * (p2_ref[0] + p2_ref[1])
  hn = _cheb_wide(h_ref, tx_ref, m, w_ref, b_ref)
  h_out_ref[...] = hn
  u_out_ref[...] = nrm * hn


def _t3w_call(h, tx, p264, norm, w, b2d):
  return pl.pallas_call(
      _t3w_body,
      grid=(NBLK,),
      in_specs=[
          pl.BlockSpec((BR, HID), lambda i: (i, 0)),
          pl.BlockSpec((BR, HID), lambda i: (i, 0)),
          pl.BlockSpec((2, BR, HID), lambda i: (0, i, 0)),
          pl.BlockSpec((BR, 1), lambda i: (i, 0)),
          pl.BlockSpec((3, HID, HID), lambda i: (0, 0, 0)),
          pl.BlockSpec((1, HID), lambda i: (0, 0)),
      ],
      out_specs=[
          pl.BlockSpec((BR, HID), lambda i: (i, 0)),
          pl.BlockSpec((BR, HID), lambda i: (i, 0)),
      ],
      out_shape=[
          jax.ShapeDtypeStruct((N_PAD, HID), jnp.float32),
          jax.ShapeDtypeStruct((N_PAD, HID), jnp.float32),
      ],
  )(h, tx, p264, norm, w, b2d)


def _t3f_body(h_ref, tx_ref, p2_ref, norm_ref, w_ref, b_ref,
              fcw_ref, fcb_ref, hsum_ref, logits_ref):
  i = pl.program_id(0)
  nrm = norm_ref[...]
  m = nrm * (p2_ref[0] + p2_ref[1])
  hn = _cheb_wide(h_ref, tx_ref, m, w_ref, b_ref)
  row = lax.broadcasted_iota(jnp.int32, (BR, 1), 0) + i * BR
  hn = jnp.where(row < N, hn, 0.0)

  @pl.when(i == 0)
  def _():
    hsum_ref[...] = jnp.zeros_like(hsum_ref)

  hsum_ref[...] += jnp.sum(hn, axis=0, keepdims=True)

  @pl.when(i == NBLK - 1)
  def _():
    hg = hsum_ref[...] * (1.0 / N)
    logits_ref[...] = jnp.dot(hg, fcw_ref[...],
                              preferred_element_type=jnp.float32) + fcb_ref[...]


def _t3f_call(h, tx, p264, norm, w, b2d, fc_w, fcb2d):
  _, logits = pl.pallas_call(
      _t3f_body,
      grid=(NBLK,),
      in_specs=[
          pl.BlockSpec((BR, HID), lambda i: (i, 0)),
          pl.BlockSpec((BR, HID), lambda i: (i, 0)),
          pl.BlockSpec((2, BR, HID), lambda i: (0, i, 0)),
          pl.BlockSpec((BR, 1), lambda i: (i, 0)),
          pl.BlockSpec((3, HID, HID), lambda i: (0, 0, 0)),
          pl.BlockSpec((1, HID), lambda i: (0, 0)),
          pl.BlockSpec((HID, OUT_F), lambda i: (0, 0)),
          pl.BlockSpec((1, OUT_F), lambda i: (0, 0)),
      ],
      out_specs=[
          pl.BlockSpec((1, HID), lambda i: (0, 0)),
          pl.BlockSpec((1, OUT_F), lambda i: (0, 0)),
      ],
      out_shape=[
          jax.ShapeDtypeStruct((1, HID), jnp.float32),
          jax.ShapeDtypeStruct((1, OUT_F), jnp.float32),
      ],
  )(h, tx, p264, norm, w, b2d, fc_w, fcb2d)
  return logits


@jax.jit
def kernel(x, edge_index, W1, b1, W2, b2, W3, b3, fc_w, fc_b):
  src = edge_index[0]
  dst = edge_index[1]
  pad_ids = lax.iota(jnp.int32, PAD)
  src_p = jnp.concatenate([src, pad_ids % 512]).reshape(E_PAD // SUB, SUB)
  dst_p = jnp.concatenate([dst, N + pad_ids % DUMP]).reshape(E_PAD // SUB, SUB)

  degp = _deg_kernel(dst_p)
  degp3 = degp.reshape(NC, N_PAD, 1)
  x_pad = jnp.pad(x, ((0, N_PAD - N), (0, 0)))
  norm, u1 = _t1_call(x_pad, degp3)

  p1 = _scatter_c1(src_p, dst_p, u1)
  tx1, u2 = _t2_call(p1, norm)
  p2 = _scatter_c1(src_p, dst_p, u2)
  h, u64 = _t3a_call(x_pad, tx1, p2, norm, W1, b1.reshape(1, HID))

  qsrc, qdst, qcnt = _part_kernel(src_p, dst_p)
  for layer in (1, 2):
    p1 = _scat64q(qsrc, qdst, qcnt, u64)
    tx, u2_64 = _t2w_call(p1, norm)
    p264 = _scat64q(qsrc, qdst, qcnt, u2_64)
    if layer == 1:
      h, u64 = _t3w_call(h, tx, p264, norm, W2, b2.reshape(1, HID))
    else:
      logits = _t3f_call(h, tx, p264, norm, W3, b3.reshape(1, HID),
                         fc_w, fc_b.reshape(1, OUT_F))
  return logits
